# baseline (speedup 1.0000x reference)
import functools

import jax
import jax.numpy as jnp
from jax import lax
from jax.experimental import pallas as pl
from jax.experimental.pallas import tpu as pltpu
from jax.experimental.pallas import tpu_sc as plsc

_NC = 2
_NS = 16
_L = 16

_CHUNK = 128
_BLK = 131072
_SLAB = 2048
_NSLAB = 6
_SPLIT = _NC * _NS * _SLAB * _NSLAB


@functools.lru_cache(maxsize=None)
def _make_tc_matvec(E, V):
    n_tc = V - _SPLIT
    blk0 = _SPLIT // _BLK
    grid = (n_tc + _BLK - 1) // _BLK

    def body(tab_ref, w_ref, z_ref):
        z_ref[...] = jnp.sum(tab_ref[...] * w_ref[...], axis=0)

    return pl.pallas_call(
        body,
        grid=(grid,),
        in_specs=[
            pl.BlockSpec((E, _BLK), lambda j: (0, blk0 + j)),
            pl.BlockSpec((E, 1), lambda j: (0, 0)),
        ],
        out_specs=pl.BlockSpec((_BLK,), lambda j: (j,)),
        out_shape=jax.ShapeDtypeStruct((n_tc,), jnp.float32),
        compiler_params=pltpu.CompilerParams(disable_bounds_checks=True),
    )


@functools.lru_cache(maxsize=None)
def _make_sc_matvec(E, V):
    NW = _NC * _NS
    per_w = _SLAB * _NSLAB
    ngroups = _SLAB // _L
    mesh = plsc.VectorSubcoreMesh(core_axis_name="c", subcore_axis_name="s")

    @functools.partial(
        pl.kernel,
        mesh=mesh,
        compiler_params=pltpu.CompilerParams(
            needs_layout_passes=False, use_tc_tiling_on_sc=True
        ),
        out_type=jax.ShapeDtypeStruct((_SPLIT,), jnp.float32),
        scratch_types=[
            pltpu.VMEM((E, _SLAB), jnp.float32),
            pltpu.VMEM((E, _SLAB), jnp.float32),
            pltpu.VMEM((_SLAB,), jnp.float32),
            pltpu.VMEM((_L,), jnp.float32),
            pltpu.VMEM((_L,), jnp.float32),
            pltpu.SemaphoreType.DMA,
            pltpu.SemaphoreType.DMA,
        ],
    )
    def sc_matvec(tab_hbm, lw_hbm, zsc_hbm,
                  slab0, slab1, zbuf, w_v, w1_v, sem0, sem1):
        wid = lax.axis_index("s") * _NC + lax.axis_index("c")
        base = wid * per_w

        pltpu.sync_copy(lw_hbm.at[pl.ds(0, _L)], w_v)
        pltpu.sync_copy(lw_hbm.at[pl.ds(_L, 1)], w1_v.at[pl.ds(0, 1)])
        w_lo = w_v[pl.ds(0, _L)]
        wks = [w_lo[k + 1] for k in range(15)] + [w1_v[pl.ds(0, _L)][0]]

        slabs = [slab0, slab1]
        sems = [sem0, sem1]
        cps = [None, None]
        cps[0] = pltpu.async_copy(
            tab_hbm.at[:, pl.ds(base, _SLAB)], slab0, sem0
        )
        for s in range(_NSLAB):
            cur = slabs[s % 2]
            if s + 1 < _NSLAB:
                cps[(s + 1) % 2] = pltpu.async_copy(
                    tab_hbm.at[:, pl.ds(base + (s + 1) * _SLAB, _SLAB)],
                    slabs[(s + 1) % 2],
                    sems[(s + 1) % 2],
                )
            cps[s % 2].wait()

            def group(g, carry):
                sl = pl.ds(g * _L, _L)
                acc = cur[0, sl] * wks[0]
                for k in range(1, E):
                    acc = acc + cur[k, sl] * wks[k]
                zbuf[sl] = acc
                return carry

            lax.fori_loop(0, ngroups, group, 0)
            pltpu.sync_copy(
                zbuf, zsc_hbm.at[pl.ds(base + s * _SLAB, _SLAB)]
            )

    return sc_matvec


@functools.lru_cache(maxsize=None)
def _make_sc_gather(B, V):
    NW = _NC * _NS
    bpw = B // NW
    nchunk = bpw // _CHUNK
    ngroups = _CHUNK // _L
    mesh = plsc.VectorSubcoreMesh(core_axis_name="c", subcore_axis_name="s")

    @functools.partial(
        pl.kernel,
        mesh=mesh,
        compiler_params=pltpu.CompilerParams(
            needs_layout_passes=False, use_tc_tiling_on_sc=False
        ),
        out_type=jax.ShapeDtypeStruct((B,), jnp.float32),
        scratch_types=[
            pltpu.VMEM((nchunk, _CHUNK), jnp.int32),
            pltpu.VMEM((nchunk, _CHUNK), jnp.int32),
            pltpu.VMEM((nchunk, _CHUNK), jnp.int32),
            pltpu.VMEM((bpw,), jnp.float32),
            pltpu.VMEM((bpw,), jnp.float32),
            pltpu.VMEM((bpw,), jnp.float32),
            pltpu.VMEM((bpw,), jnp.float32),
            pltpu.VMEM((_L,), jnp.float32),
            pltpu.VMEM((_L,), jnp.float32),
            pltpu.SemaphoreType.DMA,
            pltpu.SemaphoreType.DMA,
        ],
    )
    def sc_gather(zsc_hbm, ztc_hbm, x_hbm, c_hbm, lw_hbm, lb_hbm, out_hbm,
                  idx_v, idx1_v, idx2_v, zg1_v, zg2_v, x_v, out_v,
                  w_v, b_v, sem, sem2):
        wid = lax.axis_index("s") * _NC + lax.axis_index("c")
        base = wid * bpw

        pltpu.sync_copy(c_hbm.at[pl.ds(wid * nchunk, nchunk), :], idx_v)
        cp_x = pltpu.async_copy(x_hbm.at[pl.ds(base, bpw)], x_v, sem2)
        cp_w = pltpu.async_copy(lw_hbm.at[pl.ds(0, _L)], w_v, sem2)
        cp_b = pltpu.async_copy(lb_hbm, b_v.at[pl.ds(0, 1)], sem2)

        def make_split(j):
            def split_idx(g, carry):
                sl = pl.ds(g * _L, _L)
                iv = idx_v[j, sl]
                idx1_v[j, sl] = jnp.minimum(iv, _SPLIT - 1)
                idx2_v[j, sl] = jnp.maximum(iv - _SPLIT, 0)
                return carry
            return split_idx

        for j in range(nchunk):
            lax.fori_loop(0, ngroups, make_split(j), 0)

        copies = []
        for j in range(nchunk):
            copies.append(
                pltpu.async_copy(
                    zsc_hbm.at[idx1_v.at[j]],
                    zg1_v.at[pl.ds(j * _CHUNK, _CHUNK)],
                    sem,
                )
            )
            copies.append(
                pltpu.async_copy(
                    ztc_hbm.at[idx2_v.at[j]],
                    zg2_v.at[pl.ds(j * _CHUNK, _CHUNK)],
                    sem,
                )
            )

        cp_x.wait()
        cp_w.wait()
        cp_b.wait()
        wx = w_v[pl.ds(0, _L)][0]
        bb = b_v[pl.ds(0, _L)][0]

        def make_group(j):
            def group(g, carry):
                sl = pl.ds(j * _CHUNK + g * _L, _L)
                iv = idx_v[j, pl.ds(g * _L, _L)]
                zval = jnp.where(iv < _SPLIT, zg1_v[sl], zg2_v[sl])
                out_v[sl] = zval + x_v[sl] * wx + bb
                return carry
            return group

        for j in range(nchunk):
            copies[2 * j].wait()
            copies[2 * j + 1].wait()
            lax.fori_loop(0, ngroups, make_group(j), 0)

        pltpu.sync_copy(out_v, out_hbm.at[pl.ds(base, bpw)])

    return sc_gather


def kernel(x, c, emb_table, lin_w, lin_b):
    B = x.shape[0]
    V, E = emb_table.shape
    xf = x.reshape(B).astype(jnp.float32)
    c2d = c.astype(jnp.int32).reshape(B // _CHUNK, _CHUNK)
    w_col = lin_w[0, 1:].reshape(E, 1)
    lw17 = lin_w.reshape(E + 1)
    tabT = emb_table.T
    z_tc = _make_tc_matvec(E, V)(tabT, w_col)
    z_sc = _make_sc_matvec(E, V)(tabT, lw17)
    out = _make_sc_gather(B, V)(z_sc, z_tc, xf, c2d, lw17, lin_b)
    return out.reshape(B, 1)


# speedup vs baseline: 1.7542x; 1.7542x over previous
import functools

import jax
import jax.numpy as jnp
from jax import lax
from jax.experimental import pallas as pl
from jax.experimental.pallas import tpu as pltpu
from jax.experimental.pallas import tpu_sc as plsc

_NC = 2
_NS = 16
_L = 16

_CHUNK = 128
_BLK = 131072
_SLAB = 2048
_NSLAB = 6
_SPLIT = _NC * _NS * _SLAB * _NSLAB


@functools.lru_cache(maxsize=None)
def _make_tc_matvec(E, V):
    n_tc = V - _SPLIT
    blk0 = _SPLIT // _BLK
    grid = (n_tc + _BLK - 1) // _BLK

    def body(tab_ref, w_ref, z_ref):
        z_ref[...] = jnp.sum(tab_ref[...] * w_ref[...], axis=0)

    return pl.pallas_call(
        body,
        grid=(grid,),
        in_specs=[
            pl.BlockSpec((E, _BLK), lambda j: (0, blk0 + j)),
            pl.BlockSpec((E, 1), lambda j: (0, 0)),
        ],
        out_specs=pl.BlockSpec((_BLK,), lambda j: (j,)),
        out_shape=jax.ShapeDtypeStruct((n_tc,), jnp.float32),
        compiler_params=pltpu.CompilerParams(disable_bounds_checks=True),
    )


@functools.lru_cache(maxsize=None)
def _make_sc_matvec(E, V):
    NW = _NC * _NS
    per_w = _SLAB * _NSLAB
    ngroups = _SLAB // _L
    mesh = plsc.VectorSubcoreMesh(core_axis_name="c", subcore_axis_name="s")

    @functools.partial(
        pl.kernel,
        mesh=mesh,
        compiler_params=pltpu.CompilerParams(
            needs_layout_passes=False, use_tc_tiling_on_sc=True
        ),
        out_type=jax.ShapeDtypeStruct((_SPLIT,), jnp.float32),
        scratch_types=[
            pltpu.VMEM((E, _SLAB), jnp.float32),
            pltpu.VMEM((E, _SLAB), jnp.float32),
            pltpu.VMEM((_SLAB,), jnp.float32),
            pltpu.VMEM((_L,), jnp.float32),
            pltpu.VMEM((_L,), jnp.float32),
            pltpu.SemaphoreType.DMA,
            pltpu.SemaphoreType.DMA,
        ],
    )
    def sc_matvec(tab_hbm, lw_hbm, zsc_hbm,
                  slab0, slab1, zbuf, w_v, w1_v, sem0, sem1):
        wid = lax.axis_index("s") * _NC + lax.axis_index("c")
        base = wid * per_w

        pltpu.sync_copy(lw_hbm.at[pl.ds(0, _L)], w_v)
        pltpu.sync_copy(lw_hbm.at[pl.ds(_L, 1)], w1_v.at[pl.ds(0, 1)])
        w_lo = w_v[pl.ds(0, _L)]
        wks = [w_lo[k + 1] for k in range(15)] + [w1_v[pl.ds(0, _L)][0]]

        slabs = [slab0, slab1]
        sems = [sem0, sem1]
        cps = [None, None]
        cps[0] = pltpu.async_copy(
            tab_hbm.at[:, pl.ds(base, _SLAB)], slab0, sem0
        )
        for s in range(_NSLAB):
            cur = slabs[s % 2]
            if s + 1 < _NSLAB:
                cps[(s + 1) % 2] = pltpu.async_copy(
                    tab_hbm.at[:, pl.ds(base + (s + 1) * _SLAB, _SLAB)],
                    slabs[(s + 1) % 2],
                    sems[(s + 1) % 2],
                )
            cps[s % 2].wait()

            def group(g, carry):
                sl = pl.ds(g * _L, _L)
                acc = cur[0, sl] * wks[0]
                for k in range(1, E):
                    acc = acc + cur[k, sl] * wks[k]
                zbuf[sl] = acc
                return carry

            lax.fori_loop(0, ngroups, group, 0)
            pltpu.sync_copy(
                zbuf, zsc_hbm.at[pl.ds(base + s * _SLAB, _SLAB)]
            )

    return sc_matvec


@functools.lru_cache(maxsize=None)
def _make_sc_gather(B, V):
    NW = _NC * _NS
    bpw = B // NW
    nchunk = bpw // _CHUNK
    ngroups = _CHUNK // _L
    mesh = plsc.VectorSubcoreMesh(core_axis_name="c", subcore_axis_name="s")

    @functools.partial(
        pl.kernel,
        mesh=mesh,
        compiler_params=pltpu.CompilerParams(
            needs_layout_passes=False, use_tc_tiling_on_sc=False
        ),
        out_type=jax.ShapeDtypeStruct((B,), jnp.float32),
        scratch_types=[
            pltpu.VMEM((nchunk, _CHUNK), jnp.int32),
            pltpu.VMEM((nchunk, _CHUNK), jnp.int32),
            pltpu.VMEM((nchunk, _CHUNK), jnp.int32),
            pltpu.VMEM((bpw,), jnp.float32),
            pltpu.VMEM((bpw,), jnp.float32),
            pltpu.VMEM((bpw,), jnp.float32),
            pltpu.VMEM((bpw,), jnp.float32),
            pltpu.VMEM((_L,), jnp.float32),
            pltpu.VMEM((_L,), jnp.float32),
            pltpu.SemaphoreType.DMA,
            pltpu.SemaphoreType.DMA,
        ],
    )
    def sc_gather(zsc_hbm, ztc_hbm, x_hbm, c_hbm, lw_hbm, lb_hbm, out_hbm,
                  idx_v, idx1_v, idx2_v, zg1_v, zg2_v, x_v, out_v,
                  w_v, b_v, sem, sem2):
        wid = lax.axis_index("s") * _NC + lax.axis_index("c")
        base = wid * bpw

        pltpu.sync_copy(c_hbm.at[pl.ds(wid * nchunk, nchunk), :], idx_v)
        cp_x = pltpu.async_copy(x_hbm.at[pl.ds(base, bpw)], x_v, sem2)
        cp_w = pltpu.async_copy(lw_hbm.at[pl.ds(0, _L)], w_v, sem2)
        cp_b = pltpu.async_copy(lb_hbm, b_v.at[pl.ds(0, 1)], sem2)

        def make_split(j):
            def split_idx(g, carry):
                sl = pl.ds(g * _L, _L)
                iv = idx_v[j, sl]
                idx1_v[j, sl] = iv % _SPLIT
                idx2_v[j, sl] = jnp.where(iv >= _SPLIT, iv - _SPLIT, iv)
                return carry
            return split_idx

        for j in range(nchunk):
            lax.fori_loop(0, ngroups, make_split(j), 0)

        copies = []
        for j in range(nchunk):
            copies.append(
                pltpu.async_copy(
                    zsc_hbm.at[idx1_v.at[j]],
                    zg1_v.at[pl.ds(j * _CHUNK, _CHUNK)],
                    sem,
                )
            )
            copies.append(
                pltpu.async_copy(
                    ztc_hbm.at[idx2_v.at[j]],
                    zg2_v.at[pl.ds(j * _CHUNK, _CHUNK)],
                    sem,
                )
            )

        cp_x.wait()
        cp_w.wait()
        cp_b.wait()
        wx = w_v[pl.ds(0, _L)][0]
        bb = b_v[pl.ds(0, _L)][0]

        def make_group(j):
            def group(g, carry):
                sl = pl.ds(j * _CHUNK + g * _L, _L)
                iv = idx_v[j, pl.ds(g * _L, _L)]
                zval = jnp.where(iv < _SPLIT, zg1_v[sl], zg2_v[sl])
                out_v[sl] = zval + x_v[sl] * wx + bb
                return carry
            return group

        for j in range(nchunk):
            copies[2 * j].wait()
            copies[2 * j + 1].wait()
            lax.fori_loop(0, ngroups, make_group(j), 0)

        pltpu.sync_copy(out_v, out_hbm.at[pl.ds(base, bpw)])

    return sc_gather


def kernel(x, c, emb_table, lin_w, lin_b):
    B = x.shape[0]
    V, E = emb_table.shape
    xf = x.reshape(B).astype(jnp.float32)
    c2d = c.astype(jnp.int32).reshape(B // _CHUNK, _CHUNK)
    w_col = lin_w[0, 1:].reshape(E, 1)
    lw17 = lin_w.reshape(E + 1)
    tabT = emb_table.T
    z_tc = _make_tc_matvec(E, V)(tabT, w_col)
    z_sc = _make_sc_matvec(E, V)(tabT, lw17)
    out = _make_sc_gather(B, V)(z_sc, z_tc, xf, c2d, lw17, lin_b)
    return out.reshape(B, 1)


# speedup vs baseline: 1.9724x; 1.1244x over previous
import functools

import jax
import jax.numpy as jnp
from jax import lax
from jax.experimental import pallas as pl
from jax.experimental.pallas import tpu as pltpu
from jax.experimental.pallas import tpu_sc as plsc

_NC = 2
_NS = 16
_L = 16

_CHUNK = 128
_BLK = 131072
_SLAB = 2048
_NSLAB = 4
_SPLIT = _NC * _NS * _SLAB * _NSLAB


@functools.lru_cache(maxsize=None)
def _make_tc_matvec(E, V):
    n_tc = V - _SPLIT
    blk0 = _SPLIT // _BLK
    grid = (n_tc + _BLK - 1) // _BLK

    def body(tab_ref, w_ref, z_ref):
        z_ref[...] = jnp.sum(tab_ref[...] * w_ref[...], axis=0)

    return pl.pallas_call(
        body,
        grid=(grid,),
        in_specs=[
            pl.BlockSpec((E, _BLK), lambda j: (0, blk0 + j)),
            pl.BlockSpec((E, 1), lambda j: (0, 0)),
        ],
        out_specs=pl.BlockSpec((_BLK,), lambda j: (j,)),
        out_shape=jax.ShapeDtypeStruct((n_tc,), jnp.float32),
        compiler_params=pltpu.CompilerParams(disable_bounds_checks=True),
    )


@functools.lru_cache(maxsize=None)
def _make_sc_matvec(E, V):
    NW = _NC * _NS
    per_w = _SLAB * _NSLAB
    ngroups = _SLAB // _L
    mesh = plsc.VectorSubcoreMesh(core_axis_name="c", subcore_axis_name="s")

    @functools.partial(
        pl.kernel,
        mesh=mesh,
        compiler_params=pltpu.CompilerParams(
            needs_layout_passes=False, use_tc_tiling_on_sc=True
        ),
        out_type=jax.ShapeDtypeStruct((_SPLIT,), jnp.float32),
        scratch_types=[
            pltpu.VMEM((E, _SLAB), jnp.float32),
            pltpu.VMEM((E, _SLAB), jnp.float32),
            pltpu.VMEM((_SLAB,), jnp.float32),
            pltpu.VMEM((_L,), jnp.float32),
            pltpu.VMEM((_L,), jnp.float32),
            pltpu.SemaphoreType.DMA,
            pltpu.SemaphoreType.DMA,
        ],
    )
    def sc_matvec(tab_hbm, lw_hbm, zsc_hbm,
                  slab0, slab1, zbuf, w_v, w1_v, sem0, sem1):
        wid = lax.axis_index("s") * _NC + lax.axis_index("c")
        base = wid * per_w

        pltpu.sync_copy(lw_hbm.at[pl.ds(0, _L)], w_v)
        pltpu.sync_copy(lw_hbm.at[pl.ds(_L, 1)], w1_v.at[pl.ds(0, 1)])
        w_lo = w_v[pl.ds(0, _L)]
        wks = [w_lo[k + 1] for k in range(15)] + [w1_v[pl.ds(0, _L)][0]]

        slabs = [slab0, slab1]
        sems = [sem0, sem1]
        cps = [None, None]
        cps[0] = pltpu.async_copy(
            tab_hbm.at[:, pl.ds(base, _SLAB)], slab0, sem0
        )
        for s in range(_NSLAB):
            cur = slabs[s % 2]
            if s + 1 < _NSLAB:
                cps[(s + 1) % 2] = pltpu.async_copy(
                    tab_hbm.at[:, pl.ds(base + (s + 1) * _SLAB, _SLAB)],
                    slabs[(s + 1) % 2],
                    sems[(s + 1) % 2],
                )
            cps[s % 2].wait()

            def group(g, carry):
                for u in range(4):
                    sl = pl.ds((g * 4 + u) * _L, _L)
                    acc = cur[0, sl] * wks[0]
                    for k in range(1, E):
                        acc = acc + cur[k, sl] * wks[k]
                    zbuf[sl] = acc
                return carry

            lax.fori_loop(0, ngroups // 4, group, 0)
            pltpu.sync_copy(
                zbuf, zsc_hbm.at[pl.ds(base + s * _SLAB, _SLAB)]
            )

    return sc_matvec


@functools.lru_cache(maxsize=None)
def _make_sc_gather(B, V):
    NW = _NC * _NS
    bpw = B // NW
    nchunk = bpw // _CHUNK
    ngroups = _CHUNK // _L
    mesh = plsc.VectorSubcoreMesh(core_axis_name="c", subcore_axis_name="s")

    @functools.partial(
        pl.kernel,
        mesh=mesh,
        compiler_params=pltpu.CompilerParams(
            needs_layout_passes=False, use_tc_tiling_on_sc=False
        ),
        out_type=jax.ShapeDtypeStruct((B,), jnp.float32),
        scratch_types=[
            pltpu.VMEM((nchunk, _CHUNK), jnp.int32),
            pltpu.VMEM((nchunk, _CHUNK), jnp.int32),
            pltpu.VMEM((nchunk, _CHUNK), jnp.int32),
            pltpu.VMEM((bpw,), jnp.float32),
            pltpu.VMEM((bpw,), jnp.float32),
            pltpu.VMEM((bpw,), jnp.float32),
            pltpu.VMEM((bpw,), jnp.float32),
            pltpu.VMEM((_L,), jnp.float32),
            pltpu.VMEM((_L,), jnp.float32),
            pltpu.SemaphoreType.DMA,
            pltpu.SemaphoreType.DMA,
        ],
    )
    def sc_gather(zsc_hbm, ztc_hbm, x_hbm, c_hbm, lw_hbm, lb_hbm, out_hbm,
                  idx_v, idx1_v, idx2_v, zg1_v, zg2_v, x_v, out_v,
                  w_v, b_v, sem, sem2):
        wid = lax.axis_index("s") * _NC + lax.axis_index("c")
        base = wid * bpw

        pltpu.sync_copy(c_hbm.at[pl.ds(wid * nchunk, nchunk), :], idx_v)
        cp_x = pltpu.async_copy(x_hbm.at[pl.ds(base, bpw)], x_v, sem2)
        cp_w = pltpu.async_copy(lw_hbm.at[pl.ds(0, _L)], w_v, sem2)
        cp_b = pltpu.async_copy(lb_hbm, b_v.at[pl.ds(0, 1)], sem2)

        def make_split(j):
            def split_idx(g, carry):
                sl = pl.ds(g * _L, _L)
                iv = idx_v[j, sl]
                idx1_v[j, sl] = iv % _SPLIT
                idx2_v[j, sl] = jnp.where(iv >= _SPLIT, iv - _SPLIT, iv)
                return carry
            return split_idx

        for j in range(nchunk):
            lax.fori_loop(0, ngroups, make_split(j), 0)

        copies = []
        for j in range(nchunk):
            copies.append(
                pltpu.async_copy(
                    zsc_hbm.at[idx1_v.at[j]],
                    zg1_v.at[pl.ds(j * _CHUNK, _CHUNK)],
                    sem,
                )
            )
            copies.append(
                pltpu.async_copy(
                    ztc_hbm.at[idx2_v.at[j]],
                    zg2_v.at[pl.ds(j * _CHUNK, _CHUNK)],
                    sem,
                )
            )

        cp_x.wait()
        cp_w.wait()
        cp_b.wait()
        wx = w_v[pl.ds(0, _L)][0]
        bb = b_v[pl.ds(0, _L)][0]

        def make_group(j):
            def group(g, carry):
                sl = pl.ds(j * _CHUNK + g * _L, _L)
                iv = idx_v[j, pl.ds(g * _L, _L)]
                zval = jnp.where(iv < _SPLIT, zg1_v[sl], zg2_v[sl])
                out_v[sl] = zval + x_v[sl] * wx + bb
                return carry
            return group

        for j in range(nchunk):
            copies[2 * j].wait()
            copies[2 * j + 1].wait()
            lax.fori_loop(0, ngroups, make_group(j), 0)

        pltpu.sync_copy(out_v, out_hbm.at[pl.ds(base, bpw)])

    return sc_gather


def kernel(x, c, emb_table, lin_w, lin_b):
    B = x.shape[0]
    V, E = emb_table.shape
    xf = x.reshape(B).astype(jnp.float32)
    c2d = c.astype(jnp.int32).reshape(B // _CHUNK, _CHUNK)
    w_col = lin_w[0, 1:].reshape(E, 1)
    lw17 = lin_w.reshape(E + 1)
    tabT = emb_table.T
    z_tc = _make_tc_matvec(E, V)(tabT, w_col)
    z_sc = _make_sc_matvec(E, V)(tabT, lw17)
    out = _make_sc_gather(B, V)(z_sc, z_tc, xf, c2d, lw17, lin_b)
    return out.reshape(B, 1)


# speedup vs baseline: 2.2345x; 1.1329x over previous
import functools

import jax
import jax.numpy as jnp
from jax import lax
from jax.experimental import pallas as pl
from jax.experimental.pallas import tpu as pltpu
from jax.experimental.pallas import tpu_sc as plsc

_NC = 2
_NS = 16
_L = 16

_CHUNK = 128
_BLK = 131072


@functools.lru_cache(maxsize=None)
def _make_tc_matvec(E, V):
    grid = (V + _BLK - 1) // _BLK

    def body(tab_ref, w_ref, z_ref):
        z_ref[...] = jnp.sum(tab_ref[...] * w_ref[...], axis=0)

    return pl.pallas_call(
        body,
        grid=(grid,),
        in_specs=[
            pl.BlockSpec((E, _BLK), lambda j: (0, j)),
            pl.BlockSpec((E, 1), lambda j: (0, 0)),
        ],
        out_specs=pl.BlockSpec((_BLK,), lambda j: (j,)),
        out_shape=jax.ShapeDtypeStruct((V,), jnp.float32),
        compiler_params=pltpu.CompilerParams(disable_bounds_checks=True),
    )


@functools.lru_cache(maxsize=None)
def _make_sc_gather(B, V):
    NW = _NC * _NS
    bpw = B // NW
    nchunk = bpw // _CHUNK
    ngroups = _CHUNK // _L
    mesh = plsc.VectorSubcoreMesh(core_axis_name="c", subcore_axis_name="s")

    @functools.partial(
        pl.kernel,
        mesh=mesh,
        compiler_params=pltpu.CompilerParams(
            needs_layout_passes=False,
            use_tc_tiling_on_sc=False,
            skip_device_barrier=True,
            disable_bounds_checks=True,
            disable_semaphore_checks=True,
        ),
        out_type=jax.ShapeDtypeStruct((B,), jnp.float32),
        scratch_types=[
            pltpu.VMEM((nchunk, _CHUNK), jnp.int32),
            pltpu.VMEM((bpw,), jnp.float32),
            pltpu.VMEM((bpw,), jnp.float32),
            pltpu.VMEM((bpw,), jnp.float32),
            pltpu.VMEM((_L,), jnp.float32),
            pltpu.VMEM((_L,), jnp.float32),
            pltpu.SemaphoreType.DMA,
            pltpu.SemaphoreType.DMA,
        ],
    )
    def sc_kernel(z_hbm, x_hbm, c_hbm, lw_hbm, lb_hbm, out_hbm,
                  idx_v, zg_v, x_v, out_v, w_v, b_v, sem, sem2):
        wid = lax.axis_index("s") * _NC + lax.axis_index("c")
        base = wid * bpw

        pltpu.sync_copy(c_hbm.at[pl.ds(wid * nchunk, nchunk), :], idx_v)
        cp_x = pltpu.async_copy(x_hbm.at[pl.ds(base, bpw)], x_v, sem2)
        cp_w = pltpu.async_copy(lw_hbm.at[pl.ds(0, _L)], w_v, sem2)
        cp_b = pltpu.async_copy(lb_hbm, b_v.at[pl.ds(0, 1)], sem2)

        copies = []
        for j in range(nchunk):
            copies.append(
                pltpu.async_copy(
                    z_hbm.at[idx_v.at[j]],
                    zg_v.at[pl.ds(j * _CHUNK, _CHUNK)],
                    sem,
                )
            )

        cp_x.wait()
        cp_w.wait()
        cp_b.wait()
        wx = w_v[pl.ds(0, _L)][0]
        bb = b_v[pl.ds(0, _L)][0]

        def group(g, carry):
            sl = pl.ds(g * _L, _L)
            out_v[sl] = zg_v[sl] + x_v[sl] * wx + bb
            return carry

        for j in range(nchunk):
            copies[j].wait()
            lax.fori_loop(j * ngroups, (j + 1) * ngroups, group, 0)

        pltpu.sync_copy(out_v, out_hbm.at[pl.ds(base, bpw)])

    return sc_kernel


def kernel(x, c, emb_table, lin_w, lin_b):
    B = x.shape[0]
    V, E = emb_table.shape
    xf = x.reshape(B).astype(jnp.float32)
    c2d = c.astype(jnp.int32).reshape(B // _CHUNK, _CHUNK)
    w_col = lin_w[0, 1:].reshape(E, 1)
    lw17 = lin_w.reshape(E + 1)
    z = _make_tc_matvec(E, V)(emb_table.T, w_col)
    out = _make_sc_gather(B, V)(z, xf, c2d, lw17, lin_b)
    return out.reshape(B, 1)
